# baseline (speedup 1.0000x reference)
import jax
import jax.numpy as jnp
from jax import lax
from jax.experimental import pallas as pl
from jax.experimental.pallas import tpu as pltpu

_N, _D = 4096, 512
_TILE = 256
_GRID = _N // _TILE
_HALF = _GRID // 2
_LANES = 128


def _loss_kernel(x0, y0, x1, y1, x2, y2, x3, y3, m_ref, out_ref):
    t = pl.program_id(1)

    @pl.when(t == 0)
    def _zero():
        out_ref[...] = jnp.zeros_like(out_ref)

    lane = lax.broadcasted_iota(jnp.int32, (1, _LANES), 1)
    acc = jnp.zeros((1, _LANES), jnp.float32)
    for k, (x, y) in enumerate(((x0, y0), (x1, y1), (x2, y2), (x3, y3))):
        mk = m_ref[k:k + 1]
        d = x[...] - y[...]
        d2 = d * d
        sv = jnp.dot(mk, d2, preferred_element_type=jnp.float32)
        s = jnp.sum(sv)
        c = jnp.sum(mk)
        acc = acc + jnp.where(lane == k, s, 0.0)
        acc = acc + jnp.where(lane == k + 4, c, 0.0)
    out_ref[0] = out_ref[0] + acc


def _partials(x0, y0, x1, y1, x2, y2, x3, y3, mrow):
    xy_spec = pl.BlockSpec((_TILE, _D), lambda i, t: (i * _HALF + t, 0))
    m_spec = pl.BlockSpec((4, _TILE), lambda i, t: (0, i * _HALF + t))
    return pl.pallas_call(
        _loss_kernel,
        out_shape=jax.ShapeDtypeStruct((2, 1, _LANES), jnp.float32),
        grid=(2, _HALF),
        in_specs=[xy_spec] * 8 + [m_spec],
        out_specs=pl.BlockSpec((1, 1, _LANES), lambda i, t: (i, 0, 0)),
        compiler_params=pltpu.CompilerParams(
            dimension_semantics=("parallel", "arbitrary"),
            vmem_limit_bytes=64 * 1024 * 1024),
    )(x0, y0, x1, y1, x2, y2, x3, y3, mrow)


@jax.jit
def kernel(inputs_0, targets_0, masks_0,
           inputs_1, targets_1, masks_1,
           inputs_2, targets_2, masks_2,
           inputs_3, targets_3, masks_3):
    mrow = jnp.concatenate(
        (masks_0, masks_1, masks_2, masks_3)).reshape(4, _N).astype(
            jnp.float32)
    part = _partials(inputs_0, targets_0, inputs_1, targets_1,
                     inputs_2, targets_2, inputs_3, targets_3, mrow)
    vec = part[0, 0] + part[1, 0]
    sums = lax.slice(vec, (0,), (4,))
    counts = lax.slice(vec, (4,), (8,))
    losses = jnp.where(counts > 0, sums / jnp.maximum(counts * _D, 1.0), 0.0)
    return jnp.sum(losses)


# speedup vs baseline: 1.1278x; 1.1278x over previous
import jax
import jax.numpy as jnp
from jax import lax
from jax.experimental import pallas as pl
from jax.experimental.pallas import tpu as pltpu

_N, _D = 4096, 512
_TILE = 512
_GRID = _N // _TILE
_LANES = 128


def _loss_kernel(x0, y0, x1, y1, x2, y2, x3, y3, m_ref, out_ref):
    lane = lax.broadcasted_iota(jnp.int32, (1, _LANES), 1)
    acc = jnp.zeros((1, _LANES), jnp.float32)
    for k, (x, y) in enumerate(((x0, y0), (x1, y1), (x2, y2), (x3, y3))):
        mk = m_ref[k:k + 1]
        d = x[...] - y[...]
        d2 = d * d
        sv = jnp.dot(mk, d2, preferred_element_type=jnp.float32)
        s = jnp.sum(sv)
        c = jnp.sum(mk)
        acc = acc + jnp.where(lane == k, s, 0.0)
        acc = acc + jnp.where(lane == k + 4, c, 0.0)
    out_ref[0] = acc


def _partials(x0, y0, x1, y1, x2, y2, x3, y3, mrow):
    xy_spec = pl.BlockSpec((_TILE, _D), lambda g: (g, 0))
    m_spec = pl.BlockSpec((4, _TILE), lambda g: (0, g))
    return pl.pallas_call(
        _loss_kernel,
        out_shape=jax.ShapeDtypeStruct((_GRID, 1, _LANES), jnp.float32),
        grid=(_GRID,),
        in_specs=[xy_spec] * 8 + [m_spec],
        out_specs=pl.BlockSpec((1, 1, _LANES), lambda g: (g, 0, 0)),
        compiler_params=pltpu.CompilerParams(
            dimension_semantics=("parallel",),
            vmem_limit_bytes=64 * 1024 * 1024),
    )(x0, y0, x1, y1, x2, y2, x3, y3, mrow)


@jax.jit
def kernel(inputs_0, targets_0, masks_0,
           inputs_1, targets_1, masks_1,
           inputs_2, targets_2, masks_2,
           inputs_3, targets_3, masks_3):
    mrow = jnp.concatenate(
        (masks_0, masks_1, masks_2, masks_3)).reshape(4, _N).astype(
            jnp.float32)
    part = _partials(inputs_0, targets_0, inputs_1, targets_1,
                     inputs_2, targets_2, inputs_3, targets_3, mrow)
    vec = jnp.sum(part, axis=(0, 1))
    sums = lax.slice(vec, (0,), (4,))
    counts = lax.slice(vec, (4,), (8,))
    losses = jnp.where(counts > 0, sums / jnp.maximum(counts * _D, 1.0), 0.0)
    return jnp.sum(losses)
